# baseline (speedup 1.0000x reference)
import jax
import jax.numpy as jnp
from jax import lax
from jax.experimental import pallas as pl
from jax.experimental.pallas import tpu as pltpu
from jax.experimental.pallas import tpu_sc as plsc

B, L = 4096, 200
D = 64
NC, NS, LANES = 2, 16, 16
NW = NC * NS
ROWS_PER_W = B // NW
RAW_WORDS = L * 3
N_GROUPS = (L + LANES - 1) // LANES
SEG0, SEG1 = 128, L - 128


def _sc_body(traj_ref, t_ref, l_ref, u_ref, out_ref,
             raw_v, uix_v, lix_v, tix_v, tbuf_v, lbuf_v, ubuf_v, zbuf_v,
             sem_in0, sem_in1, sem_g, sem_out0, sem_out1):
    wid = lax.axis_index("s") * NC + lax.axis_index("c")
    row0 = wid * ROWS_PER_W
    sem_in = (sem_in0, sem_in1)
    sem_out = (sem_out0, sem_out1)

    def zero_row(i, carry):
        for c in range(2 * D // LANES):
            zbuf_v[i, pl.ds(c * LANES, LANES)] = jnp.zeros((LANES,), jnp.float32)
        return carry
    lax.fori_loop(0, L, zero_row, 0)

    lane = lax.iota(jnp.int32, 16)

    def stage_idx(b, s):
        pltpu.async_copy(traj_ref.at[pl.ds(b * RAW_WORDS, RAW_WORDS)],
                         raw_v.at[s, pl.ds(0, RAW_WORDS)], sem_in[s])

    def drain_idx(s):
        pltpu.make_async_copy(traj_ref.at[pl.ds(0, RAW_WORDS)],
                              raw_v.at[s, pl.ds(0, RAW_WORDS)], sem_in[s]).wait()

    def deinterleave(s):
        for g in range(N_GROUPS):
            triple = lane * 3 + (g * LANES * 3)
            u_i = plsc.load_gather(raw_v.at[s], [triple])
            l_i = plsc.load_gather(raw_v.at[s], [triple + 1])
            t_i = plsc.load_gather(raw_v.at[s], [triple + 2])
            t_i = lax.rem(t_i + 23, 24) + 1
            j, c = (g * LANES) // 128, (g * LANES) % 128
            uix_v[s, j, pl.ds(c, LANES)] = u_i
            lix_v[s, j, pl.ds(c, LANES)] = l_i
            tix_v[s, j, pl.ds(c, LANES)] = t_i

    def fire_gathers(s):
        handles = []
        for j, (off, seg) in enumerate(((0, SEG0), (SEG0, SEG1))):
            rows = pl.ds(off, seg)
            idx = pl.ds(0, seg)
            handles.append(pltpu.async_copy(t_ref.at[tix_v.at[s, j, idx]], tbuf_v.at[s, rows], sem_g))
            handles.append(pltpu.async_copy(l_ref.at[lix_v.at[s, j, idx]], lbuf_v.at[s, rows], sem_g))
            handles.append(pltpu.async_copy(u_ref.at[uix_v.at[s, j, idx]], ubuf_v.at[s, rows], sem_g))
        return handles

    def fire_out(b, s):
        pltpu.async_copy(tbuf_v.at[s], out_ref.at[b, :, pl.ds(0 * D, D)], sem_out[s])
        pltpu.async_copy(lbuf_v.at[s], out_ref.at[b, :, pl.ds(1 * D, D)], sem_out[s])
        pltpu.async_copy(ubuf_v.at[s], out_ref.at[b, :, pl.ds(2 * D, D)], sem_out[s])
        pltpu.async_copy(zbuf_v, out_ref.at[b, :, pl.ds(3 * D, 2 * D)], sem_out[s])

    def drain_out(s):
        pltpu.make_async_copy(tbuf_v.at[s], out_ref.at[0, :, pl.ds(0 * D, D)], sem_out[s]).wait()
        pltpu.make_async_copy(lbuf_v.at[s], out_ref.at[0, :, pl.ds(1 * D, D)], sem_out[s]).wait()
        pltpu.make_async_copy(ubuf_v.at[s], out_ref.at[0, :, pl.ds(2 * D, D)], sem_out[s]).wait()
        pltpu.make_async_copy(zbuf_v, out_ref.at[0, :, pl.ds(3 * D, 2 * D)], sem_out[s]).wait()

    def step(i, s, not_first):
        b = row0 + i
        drain_idx(s)
        deinterleave(s)

        @pl.when(not_first)
        def _():
            drain_out(s)

        handles = fire_gathers(s)
        nxt = row0 + jnp.minimum(i + 1, ROWS_PER_W - 1)
        stage_idx(nxt, 1 - s)
        for h in handles:
            h.wait()
        fire_out(b, s)

    stage_idx(row0, 0)

    def pair_body(k, carry):
        step(2 * k, 0, k >= 1)
        step(2 * k + 1, 1, k >= 1)
        return carry
    lax.fori_loop(0, ROWS_PER_W // 2, pair_body, 0)

    drain_idx(0)
    drain_out(0)
    drain_out(1)


def _multi_embed(traj_flat, embed_t_w, embed_l_w, embed_u_w):
    fn = pl.kernel(
        _sc_body,
        out_type=jax.ShapeDtypeStruct((B, L, 5 * D), jnp.float32),
        mesh=plsc.VectorSubcoreMesh(core_axis_name="c", subcore_axis_name="s"),
        compiler_params=pltpu.CompilerParams(
            use_tc_tiling_on_sc=False, needs_layout_passes=False),
        scratch_types=[
            pltpu.VMEM((2, 3 * 16 * N_GROUPS), jnp.int32),
            pltpu.VMEM((2, 2, 128), jnp.int32),
            pltpu.VMEM((2, 2, 128), jnp.int32),
            pltpu.VMEM((2, 2, 128), jnp.int32),
            pltpu.VMEM((2, L, D), jnp.float32),
            pltpu.VMEM((2, L, D), jnp.float32),
            pltpu.VMEM((2, L, D), jnp.float32),
            pltpu.VMEM((L, 2 * D), jnp.float32),
            pltpu.SemaphoreType.DMA,
            pltpu.SemaphoreType.DMA,
            pltpu.SemaphoreType.DMA,
            pltpu.SemaphoreType.DMA,
            pltpu.SemaphoreType.DMA,
        ],
    )
    return fn(traj_flat, embed_t_w, embed_l_w, embed_u_w)


def kernel(trajectories, embed_t_w, embed_l_w, embed_u_w):
    traj_flat = trajectories.reshape(-1)
    return _multi_embed(traj_flat, embed_t_w, embed_l_w, embed_u_w)


# speedup vs baseline: 1.4451x; 1.4451x over previous
import jax
import jax.numpy as jnp
from jax import lax
from jax.experimental import pallas as pl
from jax.experimental.pallas import tpu as pltpu
from jax.experimental.pallas import tpu_sc as plsc

B, L = 4096, 200
D = 64
NC, NS, LANES = 2, 16, 16
NW = NC * NS
ROWS_PER_W = B // NW
SEG0, SEG1 = 128, L - 128
N_TOK = B * L


def _sc_body(uix_hbm, lix_hbm, traw_hbm, t_ref, l_ref, u_ref, out_ref,
             uix_v, lix_v, tix_v, tbuf_v, lbuf_v, ubuf_v, zbuf_v,
             sem_in0, sem_in1, sem_g, sem_out0, sem_out1):
    wid = lax.axis_index("s") * NC + lax.axis_index("c")
    row0 = wid * ROWS_PER_W
    sem_in = (sem_in0, sem_in1)
    sem_out = (sem_out0, sem_out1)

    def zero_row(i, carry):
        for c in range(2 * D // LANES):
            zbuf_v[i, pl.ds(c * LANES, LANES)] = jnp.zeros((LANES,), jnp.float32)
        return carry
    lax.fori_loop(0, L, zero_row, 0)

    def stage_idx(b, s):
        tok0 = b * L
        for src, dst in ((uix_hbm, uix_v), (lix_hbm, lix_v), (traw_hbm, tix_v)):
            pltpu.async_copy(src.at[pl.ds(tok0, SEG0)], dst.at[s, 0], sem_in[s])
            pltpu.async_copy(src.at[pl.ds(tok0 + SEG0, SEG1)],
                             dst.at[s, 1, pl.ds(0, SEG1)], sem_in[s])

    def drain_idx(s):
        for dst in (uix_v, lix_v, tix_v):
            pltpu.make_async_copy(uix_hbm.at[pl.ds(0, SEG0)], dst.at[s, 0], sem_in[s]).wait()
            pltpu.make_async_copy(uix_hbm.at[pl.ds(0, SEG1)],
                                  dst.at[s, 1, pl.ds(0, SEG1)], sem_in[s]).wait()

    def remap_time(s):
        for j, ng in ((0, SEG0 // LANES), (1, (SEG1 + LANES - 1) // LANES)):
            for c in range(ng):
                sl = pl.ds(c * LANES, LANES)
                t_i = tix_v[s, j, sl]
                tix_v[s, j, sl] = lax.rem(t_i + 23, 24) + 1

    def fire_gathers(s):
        handles = []
        for j, (off, seg) in enumerate(((0, SEG0), (SEG0, SEG1))):
            rows = pl.ds(off, seg)
            idx = pl.ds(0, seg)
            handles.append(pltpu.async_copy(t_ref.at[tix_v.at[s, j, idx]], tbuf_v.at[s, rows], sem_g))
            handles.append(pltpu.async_copy(l_ref.at[lix_v.at[s, j, idx]], lbuf_v.at[s, rows], sem_g))
            handles.append(pltpu.async_copy(u_ref.at[uix_v.at[s, j, idx]], ubuf_v.at[s, rows], sem_g))
        return handles

    def fire_out(b, s):
        pltpu.async_copy(tbuf_v.at[s], out_ref.at[b, :, pl.ds(0 * D, D)], sem_out[s])
        pltpu.async_copy(lbuf_v.at[s], out_ref.at[b, :, pl.ds(1 * D, D)], sem_out[s])
        pltpu.async_copy(ubuf_v.at[s], out_ref.at[b, :, pl.ds(2 * D, D)], sem_out[s])
        pltpu.async_copy(zbuf_v, out_ref.at[b, :, pl.ds(3 * D, 2 * D)], sem_out[s])

    def drain_out(s):
        pltpu.make_async_copy(tbuf_v.at[s], out_ref.at[0, :, pl.ds(0 * D, D)], sem_out[s]).wait()
        pltpu.make_async_copy(lbuf_v.at[s], out_ref.at[0, :, pl.ds(1 * D, D)], sem_out[s]).wait()
        pltpu.make_async_copy(ubuf_v.at[s], out_ref.at[0, :, pl.ds(2 * D, D)], sem_out[s]).wait()
        pltpu.make_async_copy(zbuf_v, out_ref.at[0, :, pl.ds(3 * D, 2 * D)], sem_out[s]).wait()

    def step(i, s, not_first):
        b = row0 + i
        drain_idx(s)
        remap_time(s)

        @pl.when(not_first)
        def _():
            drain_out(s)

        handles = fire_gathers(s)
        nxt = row0 + jnp.minimum(i + 1, ROWS_PER_W - 1)
        stage_idx(nxt, 1 - s)
        for h in handles:
            h.wait()
        fire_out(b, s)

    stage_idx(row0, 0)

    def pair_body(k, carry):
        step(2 * k, 0, k >= 1)
        step(2 * k + 1, 1, k >= 1)
        return carry
    lax.fori_loop(0, ROWS_PER_W // 2, pair_body, 0)

    drain_idx(0)
    drain_out(0)
    drain_out(1)


def _multi_embed(u_idx, l_idx, t_raw, embed_t_w, embed_l_w, embed_u_w):
    fn = pl.kernel(
        _sc_body,
        out_type=jax.ShapeDtypeStruct((B, L, 5 * D), jnp.float32),
        mesh=plsc.VectorSubcoreMesh(core_axis_name="c", subcore_axis_name="s"),
        compiler_params=pltpu.CompilerParams(use_tc_tiling_on_sc=False),
        scratch_types=[
            pltpu.VMEM((2, 2, 128), jnp.int32),
            pltpu.VMEM((2, 2, 128), jnp.int32),
            pltpu.VMEM((2, 2, 128), jnp.int32),
            pltpu.VMEM((2, L, D), jnp.float32),
            pltpu.VMEM((2, L, D), jnp.float32),
            pltpu.VMEM((2, L, D), jnp.float32),
            pltpu.VMEM((L, 2 * D), jnp.float32),
            pltpu.SemaphoreType.DMA,
            pltpu.SemaphoreType.DMA,
            pltpu.SemaphoreType.DMA,
            pltpu.SemaphoreType.DMA,
            pltpu.SemaphoreType.DMA,
        ],
    )
    return fn(u_idx, l_idx, t_raw, embed_t_w, embed_l_w, embed_u_w)


def kernel(trajectories, embed_t_w, embed_l_w, embed_u_w):
    flat = trajectories.reshape(N_TOK, 3)
    u_idx = flat[:, 0]
    l_idx = flat[:, 1]
    t_raw = flat[:, 2]
    return _multi_embed(u_idx, l_idx, t_raw, embed_t_w, embed_l_w, embed_u_w)


# speedup vs baseline: 2.6105x; 1.8065x over previous
import jax
import jax.numpy as jnp
from jax import lax
from jax.experimental import pallas as pl
from jax.experimental.pallas import tpu as pltpu
from jax.experimental.pallas import tpu_sc as plsc

B, L = 4096, 200
D = 64
NC, NS, LANES = 2, 16, 16
NW = NC * NS
ROWS_PER_W = B // NW
SEG0, SEG1 = 128, L - 128
N_TOK = B * L


def _sc_body(uix_hbm, lix_hbm, traw_hbm, t_ref, l_ref, u_ref, out_ref,
             uix_v, lix_v, tix_v, tbuf_v, lbuf_v, ubuf_v, zbuf_v,
             sem_in0, sem_in1, sem_g, sem_out0, sem_out1):
    wid = lax.axis_index("s") * NC + lax.axis_index("c")
    row0 = wid * ROWS_PER_W
    sem_in = (sem_in0, sem_in1)
    sem_out = (sem_out0, sem_out1)

    def zero_row(i, carry):
        for c in range(2 * D // LANES):
            zbuf_v[i, pl.ds(c * LANES, LANES)] = jnp.zeros((LANES,), jnp.float32)
        return carry
    lax.fori_loop(0, L, zero_row, 0)

    def stage_idx(b, s):
        tok0 = b * L
        for src, dst in ((uix_hbm, uix_v), (lix_hbm, lix_v), (traw_hbm, tix_v)):
            pltpu.async_copy(src.at[pl.ds(tok0, SEG0)], dst.at[s, 0], sem_in[s])
            pltpu.async_copy(src.at[pl.ds(tok0 + SEG0, SEG1)],
                             dst.at[s, 1, pl.ds(0, SEG1)], sem_in[s])

    def drain_idx(s):
        for dst in (uix_v, lix_v, tix_v):
            pltpu.make_async_copy(uix_hbm.at[pl.ds(0, SEG0)], dst.at[s, 0], sem_in[s]).wait()
            pltpu.make_async_copy(uix_hbm.at[pl.ds(0, SEG1)],
                                  dst.at[s, 1, pl.ds(0, SEG1)], sem_in[s]).wait()

    def remap_time(s):
        for j, ng in ((0, SEG0 // LANES), (1, (SEG1 + LANES - 1) // LANES)):
            for c in range(ng):
                sl = pl.ds(c * LANES, LANES)
                t_i = tix_v[s, j, sl]
                tix_v[s, j, sl] = lax.rem(t_i + 23, 24) + 1

    def fire_gathers(s):
        handles = []
        for j, (off, seg) in enumerate(((0, SEG0), (SEG0, SEG1))):
            rows = pl.ds(off, seg)
            idx = pl.ds(0, seg)
            handles.append(pltpu.async_copy(t_ref.at[tix_v.at[s, j, idx]], tbuf_v.at[s, rows], sem_g))
            handles.append(pltpu.async_copy(l_ref.at[lix_v.at[s, j, idx]], lbuf_v.at[s, rows], sem_g))
            handles.append(pltpu.async_copy(u_ref.at[uix_v.at[s, j, idx]], ubuf_v.at[s, rows], sem_g))
        return handles

    def fire_out(b, s):
        pltpu.async_copy(tbuf_v.at[s], out_ref.at[b, :, pl.ds(0 * D, D)], sem_out[s])
        pltpu.async_copy(lbuf_v.at[s], out_ref.at[b, :, pl.ds(1 * D, D)], sem_out[s])
        pltpu.async_copy(ubuf_v.at[s], out_ref.at[b, :, pl.ds(2 * D, D)], sem_out[s])
        pltpu.async_copy(zbuf_v, out_ref.at[b, :, pl.ds(3 * D, 2 * D)], sem_out[s])

    def drain_out(s):
        pltpu.make_async_copy(tbuf_v.at[s], out_ref.at[0, :, pl.ds(0 * D, D)], sem_out[s]).wait()
        pltpu.make_async_copy(lbuf_v.at[s], out_ref.at[0, :, pl.ds(1 * D, D)], sem_out[s]).wait()
        pltpu.make_async_copy(ubuf_v.at[s], out_ref.at[0, :, pl.ds(2 * D, D)], sem_out[s]).wait()
        pltpu.make_async_copy(zbuf_v, out_ref.at[0, :, pl.ds(3 * D, 2 * D)], sem_out[s]).wait()

    def step(i, s, not_first):
        b = row0 + i
        drain_idx(s)
        remap_time(s)

        @pl.when(not_first)
        def _():
            drain_out(s)

        nxt = row0 + jnp.minimum(i + 1, ROWS_PER_W - 1)
        stage_idx(nxt, 1 - s)
        fire_out(b, s)

    stage_idx(row0, 0)

    def pair_body(k, carry):
        step(2 * k, 0, k >= 1)
        step(2 * k + 1, 1, k >= 1)
        return carry
    lax.fori_loop(0, ROWS_PER_W // 2, pair_body, 0)

    drain_idx(0)
    drain_out(0)
    drain_out(1)


def _multi_embed(u_idx, l_idx, t_raw, embed_t_w, embed_l_w, embed_u_w):
    fn = pl.kernel(
        _sc_body,
        out_type=jax.ShapeDtypeStruct((B, L, 5 * D), jnp.float32),
        mesh=plsc.VectorSubcoreMesh(core_axis_name="c", subcore_axis_name="s"),
        compiler_params=pltpu.CompilerParams(use_tc_tiling_on_sc=False),
        scratch_types=[
            pltpu.VMEM((2, 2, 128), jnp.int32),
            pltpu.VMEM((2, 2, 128), jnp.int32),
            pltpu.VMEM((2, 2, 128), jnp.int32),
            pltpu.VMEM((2, L, D), jnp.float32),
            pltpu.VMEM((2, L, D), jnp.float32),
            pltpu.VMEM((2, L, D), jnp.float32),
            pltpu.VMEM((L, 2 * D), jnp.float32),
            pltpu.SemaphoreType.DMA,
            pltpu.SemaphoreType.DMA,
            pltpu.SemaphoreType.DMA,
            pltpu.SemaphoreType.DMA,
            pltpu.SemaphoreType.DMA,
        ],
    )
    return fn(u_idx, l_idx, t_raw, embed_t_w, embed_l_w, embed_u_w)


def kernel(trajectories, embed_t_w, embed_l_w, embed_u_w):
    flat = trajectories.reshape(N_TOK, 3)
    u_idx = flat[:, 0]
    l_idx = flat[:, 1]
    t_raw = flat[:, 2]
    return _multi_embed(u_idx, l_idx, t_raw, embed_t_w, embed_l_w, embed_u_w)
